# baseline (speedup 1.0000x reference)
import functools
from math import pi as PI

import jax
import jax.numpy as jnp
from jax import lax
from jax.experimental import pallas as pl
from jax.experimental.pallas import tpu as pltpu
from jax.experimental.pallas import tpu_sc as plsc

CUTOFF = 10.0
LOG2 = 0.6931471805599453


def _ssp(v):
    return jax.nn.softplus(v) - LOG2


def _lin1_body(x_ref, w_ref, o_ref):
    o_ref[...] = lax.dot_general(
        x_ref[...], w_ref[...], (((1,), (1,)), ((), ())),
        preferred_element_type=jnp.float32)


def _filter_body(attr_ref, ew_ref, w0_ref, b0_ref, w1_ref, b1_ref, o_ref):
    ht = lax.dot_general(
        w0_ref[...], attr_ref[...], (((1,), (0,)), ((), ())),
        preferred_element_type=jnp.float32) + b0_ref[...]
    ht = _ssp(ht)
    ht = lax.dot_general(
        w1_ref[...], ht, (((1,), (0,)), ((), ())),
        preferred_element_type=jnp.float32) + b1_ref[...]
    c = 0.5 * (jnp.cos(ew_ref[...] * (PI / CUTOFF)) + 1.0)
    ht = ht * c.reshape(1, c.shape[-1])
    f = ht.shape[0] // 2
    a16 = lax.bitcast_convert_type(ht[:f].astype(jnp.bfloat16), jnp.uint16)
    b16 = lax.bitcast_convert_type(ht[f:].astype(jnp.bfloat16), jnp.uint16)
    packed = a16.astype(jnp.int32) | (b16.astype(jnp.int32) << 16)
    o_ref[...] = packed.T


def _final_body(p0_ref, p1_ref, p2_ref, p3_ref, p4_ref, w_ref, b_ref, o_ref):
    agg = ((p0_ref[0] + p0_ref[1]) + (p1_ref[0] + p1_ref[1])
           + (p2_ref[0] + p2_ref[1]) + (p3_ref[0] + p3_ref[1])
           + (p4_ref[0] + p4_ref[1]))
    out = lax.dot_general(
        agg, w_ref[...], (((1,), (1,)), ((), ())),
        preferred_element_type=jnp.float32) + b_ref[...]
    o_ref[...] = _ssp(out)


def _sc_gather_mul_scatter(xl, w, src, dst, zeros, K):
    N, H = xl.shape
    n_chunks = src.shape[0] // K
    NW = 32
    iters = (n_chunks + NW - 1) // NW
    CH = (N // 16) // 8 * 8
    TAIL = N - 16 * CH

    mesh = plsc.VectorSubcoreMesh(core_axis_name="c", subcore_axis_name="s")

    @functools.partial(
        pl.kernel, mesh=mesh,
        out_type=jax.ShapeDtypeStruct((2, N, H), jnp.float32),
        scratch_types=[
            pltpu.VMEM((4, 2, K), jnp.int32),
            pltpu.VMEM((2, K, H), jnp.float32),
            pltpu.VMEM((2, K, H // 2), jnp.int32),
            pltpu.VMEM_SHARED((N, H), jnp.float32),
            pltpu.SemaphoreType.DMA((2,)),
            pltpu.SemaphoreType.DMA((2,)),
            pltpu.SemaphoreType.DMA((4,)),
        ],
    )
    def k(xl_hbm, w_hbm, src_hbm, dst_hbm, z_hbm, out_hbm,
          sd_v, rows_v, w_v, acc_s, g_sem, w_sem, i_sem):
        cid = lax.axis_index("c")
        sid = lax.axis_index("s")
        wid = cid * 16 + sid

        pltpu.sync_copy(z_hbm.at[pl.ds(sid * CH, CH)],
                        acc_s.at[pl.ds(sid * CH, CH)])
        if TAIL:
            @pl.when(sid == 0)
            def _():
                pltpu.sync_copy(z_hbm.at[pl.ds(16 * CH, TAIL)],
                                acc_s.at[pl.ds(16 * CH, TAIL)])
        plsc.subcore_barrier()

        def issue_idx(i, s):
            c = i * NW + wid

            @pl.when(c < n_chunks)
            def _():
                pltpu.async_copy(src_hbm.at[pl.ds(c * K, K)],
                                 sd_v.at[s, 0], i_sem.at[s])
                pltpu.async_copy(dst_hbm.at[pl.ds(c * K, K)],
                                 sd_v.at[s, 1], i_sem.at[s])

        def issue_main(i, b, s):
            c = i * NW + wid

            @pl.when(c < n_chunks)
            def _():
                pltpu.make_async_copy(src_hbm.at[pl.ds(0, K)], sd_v.at[s, 0],
                                      i_sem.at[s]).wait()
                pltpu.make_async_copy(src_hbm.at[pl.ds(0, K)], sd_v.at[s, 1],
                                      i_sem.at[s]).wait()
                pltpu.async_copy(w_hbm.at[pl.ds(c * K, K)], w_v.at[b],
                                 w_sem.at[b])
                pltpu.async_copy(xl_hbm.at[sd_v.at[s, 0]], rows_v.at[b],
                                 g_sem.at[b])

        def process(i, b, s):
            c = i * NW + wid

            @pl.when(c < n_chunks)
            def _():
                pltpu.make_async_copy(w_hbm.at[pl.ds(0, K)], w_v.at[b],
                                      w_sem.at[b]).wait()
                pltpu.make_async_copy(xl_hbm.at[pl.ds(0, K)], rows_v.at[b],
                                      g_sem.at[b]).wait()

                def mul_rows(r2, carry2):
                    for dr in range(2):
                        r = r2 * 2 + dr
                        for j in range(H // 32):
                            wi = w_v[b, r, pl.ds(j * 16, 16)]
                            wlo = lax.bitcast_convert_type(
                                wi << 16, jnp.float32)
                            whi = lax.bitcast_convert_type(
                                wi & jnp.int32(-65536), jnp.float32)
                            s0 = pl.ds(j * 16, 16)
                            s1 = pl.ds(H // 2 + j * 16, 16)
                            rows_v[b, r, s0] = rows_v[b, r, s0] * wlo
                            rows_v[b, r, s1] = rows_v[b, r, s1] * whi
                    return carry2

                lax.fori_loop(0, K // 2, mul_rows, 0)
                pltpu.sync_copy(rows_v.at[b], acc_s.at[sd_v.at[s, 1]],
                                add=True)

        issue_idx(0, 0)
        issue_idx(1, 1)
        issue_main(0, 0, 0)

        def outer(t, carry):
            io = t * 4
            for q in range(4):
                i = io + q
                issue_idx(i + 2, (q + 2) % 4)
                issue_main(i + 1, (q + 1) % 2, (q + 1) % 4)
                process(i, q % 2, q)
            return carry

        lax.fori_loop(0, (iters + 3) // 4, outer, 0)
        plsc.subcore_barrier()
        pltpu.sync_copy(acc_s.at[pl.ds(sid * CH, CH)],
                        out_hbm.at[cid, pl.ds(sid * CH, CH)])
        if TAIL:
            @pl.when(sid == 0)
            def _():
                pltpu.sync_copy(acc_s.at[pl.ds(16 * CH, TAIL)],
                                out_hbm.at[cid, pl.ds(16 * CH, TAIL)])

    return k(xl, w, src, dst, zeros)


def kernel(x, edge_index, edge_weight, edge_attr,
           mlp_w0, mlp_b0, mlp_w1, mlp_b1, lin1_w, lin2_w, lin2_b):
    N, H = x.shape
    E, G = edge_attr.shape
    Fq = mlp_w0.shape[0]

    K = 80
    b2r = lin2_b.reshape(1, H)
    attr_t = edge_attr.T

    BE = 3200
    b0c = mlp_b0.reshape(Fq, 1)
    b1c = mlp_b1.reshape(Fq, 1)

    xl = pl.pallas_call(
        _lin1_body,
        out_shape=jax.ShapeDtypeStruct((N, Fq), jnp.float32),
    )(x, lin1_w)

    zeros = jnp.zeros((N, Fq), jnp.float32)

    NCHUNK = 5
    EH = E // NCHUNK
    partials = []
    for h in range(NCHUNK):
        lo = h * EH
        ewh = lax.slice_in_dim(edge_weight, lo, lo + EH).reshape(
            EH // BE, 1, BE)
        attr_h = lax.slice_in_dim(attr_t, lo, lo + EH, axis=1)
        w_f = pl.pallas_call(
            _filter_body,
            grid=(EH // BE,),
            in_specs=[
                pl.BlockSpec((G, BE), lambda i: (0, i)),
                pl.BlockSpec((1, 1, BE), lambda i: (i, 0, 0)),
                pl.BlockSpec((Fq, G), lambda i: (0, 0)),
                pl.BlockSpec((Fq, 1), lambda i: (0, 0)),
                pl.BlockSpec((Fq, Fq), lambda i: (0, 0)),
                pl.BlockSpec((Fq, 1), lambda i: (0, 0)),
            ],
            out_specs=pl.BlockSpec((BE, Fq // 2), lambda i: (i, 0)),
            out_shape=jax.ShapeDtypeStruct((EH, Fq // 2), jnp.int32),
        )(attr_h, ewh, mlp_w0, b0c, mlp_w1, b1c)
        src_h = lax.slice_in_dim(edge_index[0], lo, lo + EH)
        dst_h = lax.slice_in_dim(edge_index[1], lo, lo + EH)
        partials.append(
            _sc_gather_mul_scatter(xl, w_f, src_h, dst_h, zeros, K))

    out = pl.pallas_call(
        _final_body,
        out_shape=jax.ShapeDtypeStruct((N, H), jnp.float32),
    )(*partials, lin2_w, b2r)
    return out


# speedup vs baseline: 1.0289x; 1.0289x over previous
import functools
from math import pi as PI

import jax
import jax.numpy as jnp
from jax import lax
from jax.experimental import pallas as pl
from jax.experimental.pallas import tpu as pltpu
from jax.experimental.pallas import tpu_sc as plsc

CUTOFF = 10.0
LOG2 = 0.6931471805599453


def _ssp(v):
    return jax.nn.softplus(v) - LOG2


def _lin1_body(x_ref, w_ref, o_ref):
    o_ref[...] = lax.dot_general(
        x_ref[...], w_ref[...], (((1,), (1,)), ((), ())),
        preferred_element_type=jnp.float32)


def _filter_body(attr_ref, ew_ref, w0_ref, b0_ref, w1_ref, b1_ref, o_ref):
    ht = lax.dot_general(
        w0_ref[...], attr_ref[...], (((1,), (0,)), ((), ())),
        preferred_element_type=jnp.float32) + b0_ref[...]
    ht = _ssp(ht)
    ht = lax.dot_general(
        w1_ref[...], ht, (((1,), (0,)), ((), ())),
        preferred_element_type=jnp.float32) + b1_ref[...]
    c = 0.5 * (jnp.cos(ew_ref[...] * (PI / CUTOFF)) + 1.0)
    ht = ht * c.reshape(1, c.shape[-1])
    f = ht.shape[0] // 2
    a16 = lax.bitcast_convert_type(ht[:f].astype(jnp.bfloat16), jnp.uint16)
    b16 = lax.bitcast_convert_type(ht[f:].astype(jnp.bfloat16), jnp.uint16)
    packed = a16.astype(jnp.int32) | (b16.astype(jnp.int32) << 16)
    o_ref[...] = packed.T


def _final_body(p0_ref, p1_ref, p2_ref, p3_ref, w_ref, b_ref, o_ref):
    agg = ((p0_ref[0] + p0_ref[1]) + (p1_ref[0] + p1_ref[1])
           + (p2_ref[0] + p2_ref[1]) + (p3_ref[0] + p3_ref[1]))
    out = lax.dot_general(
        agg, w_ref[...], (((1,), (1,)), ((), ())),
        preferred_element_type=jnp.float32) + b_ref[...]
    o_ref[...] = _ssp(out)


def _sc_gather_mul_scatter(xl, w, src, dst, zeros, K):
    N, H = xl.shape
    n_chunks = src.shape[0] // K
    NW = 32
    iters = (n_chunks + NW - 1) // NW
    CH = (N // 16) // 8 * 8
    TAIL = N - 16 * CH

    mesh = plsc.VectorSubcoreMesh(core_axis_name="c", subcore_axis_name="s")

    @functools.partial(
        pl.kernel, mesh=mesh,
        out_type=jax.ShapeDtypeStruct((2, N, H), jnp.float32),
        scratch_types=[
            pltpu.VMEM((4, 2, K), jnp.int32),
            pltpu.VMEM((2, K, H), jnp.float32),
            pltpu.VMEM((2, K, H // 2), jnp.int32),
            pltpu.VMEM_SHARED((N, H), jnp.float32),
            pltpu.SemaphoreType.DMA((2,)),
            pltpu.SemaphoreType.DMA((2,)),
            pltpu.SemaphoreType.DMA((4,)),
        ],
    )
    def k(xl_hbm, w_hbm, src_hbm, dst_hbm, z_hbm, out_hbm,
          sd_v, rows_v, w_v, acc_s, g_sem, w_sem, i_sem):
        cid = lax.axis_index("c")
        sid = lax.axis_index("s")
        wid = cid * 16 + sid

        pltpu.sync_copy(z_hbm.at[pl.ds(sid * CH, CH)],
                        acc_s.at[pl.ds(sid * CH, CH)])
        if TAIL:
            @pl.when(sid == 0)
            def _():
                pltpu.sync_copy(z_hbm.at[pl.ds(16 * CH, TAIL)],
                                acc_s.at[pl.ds(16 * CH, TAIL)])
        plsc.subcore_barrier()

        def issue_idx(i, s):
            c = i * NW + wid

            @pl.when(c < n_chunks)
            def _():
                pltpu.async_copy(src_hbm.at[pl.ds(c * K, K)],
                                 sd_v.at[s, 0], i_sem.at[s])
                pltpu.async_copy(dst_hbm.at[pl.ds(c * K, K)],
                                 sd_v.at[s, 1], i_sem.at[s])

        def issue_main(i, b, s):
            c = i * NW + wid

            @pl.when(c < n_chunks)
            def _():
                pltpu.make_async_copy(src_hbm.at[pl.ds(0, K)], sd_v.at[s, 0],
                                      i_sem.at[s]).wait()
                pltpu.make_async_copy(src_hbm.at[pl.ds(0, K)], sd_v.at[s, 1],
                                      i_sem.at[s]).wait()
                pltpu.async_copy(w_hbm.at[pl.ds(c * K, K)], w_v.at[b],
                                 w_sem.at[b])
                pltpu.async_copy(xl_hbm.at[sd_v.at[s, 0]], rows_v.at[b],
                                 g_sem.at[b])

        def process(i, b, s):
            c = i * NW + wid

            @pl.when(c < n_chunks)
            def _():
                pltpu.make_async_copy(w_hbm.at[pl.ds(0, K)], w_v.at[b],
                                      w_sem.at[b]).wait()
                pltpu.make_async_copy(xl_hbm.at[pl.ds(0, K)], rows_v.at[b],
                                      g_sem.at[b]).wait()

                def mul_rows(r2, carry2):
                    for dr in range(2):
                        r = r2 * 2 + dr
                        for j in range(H // 32):
                            wi = w_v[b, r, pl.ds(j * 16, 16)]
                            wlo = lax.bitcast_convert_type(
                                wi << 16, jnp.float32)
                            whi = lax.bitcast_convert_type(
                                wi & jnp.int32(-65536), jnp.float32)
                            s0 = pl.ds(j * 16, 16)
                            s1 = pl.ds(H // 2 + j * 16, 16)
                            rows_v[b, r, s0] = rows_v[b, r, s0] * wlo
                            rows_v[b, r, s1] = rows_v[b, r, s1] * whi
                    return carry2

                lax.fori_loop(0, K // 2, mul_rows, 0)
                pltpu.sync_copy(rows_v.at[b], acc_s.at[sd_v.at[s, 1]],
                                add=True)

        issue_idx(0, 0)
        issue_idx(1, 1)
        issue_main(0, 0, 0)

        def outer(t, carry):
            io = t * 4
            for q in range(4):
                i = io + q
                issue_idx(i + 2, (q + 2) % 4)
                issue_main(i + 1, (q + 1) % 2, (q + 1) % 4)
                process(i, q % 2, q)
            return carry

        lax.fori_loop(0, (iters + 3) // 4, outer, 0)
        plsc.subcore_barrier()
        pltpu.sync_copy(acc_s.at[pl.ds(sid * CH, CH)],
                        out_hbm.at[cid, pl.ds(sid * CH, CH)])
        if TAIL:
            @pl.when(sid == 0)
            def _():
                pltpu.sync_copy(acc_s.at[pl.ds(16 * CH, TAIL)],
                                out_hbm.at[cid, pl.ds(16 * CH, TAIL)])

    return k(xl, w, src, dst, zeros)


def kernel(x, edge_index, edge_weight, edge_attr,
           mlp_w0, mlp_b0, mlp_w1, mlp_b1, lin1_w, lin2_w, lin2_b):
    N, H = x.shape
    E, G = edge_attr.shape
    Fq = mlp_w0.shape[0]

    K = 80
    b2r = lin2_b.reshape(1, H)
    attr_t = edge_attr.T

    BE = 3200
    b0c = mlp_b0.reshape(Fq, 1)
    b1c = mlp_b1.reshape(Fq, 1)

    xl = pl.pallas_call(
        _lin1_body,
        out_shape=jax.ShapeDtypeStruct((N, Fq), jnp.float32),
    )(x, lin1_w)

    zeros = jnp.zeros((N, Fq), jnp.float32)

    NCHUNK = 4
    EH = E // NCHUNK
    partials = []
    for h in range(NCHUNK):
        lo = h * EH
        ewh = lax.slice_in_dim(edge_weight, lo, lo + EH).reshape(
            EH // BE, 1, BE)
        attr_h = lax.slice_in_dim(attr_t, lo, lo + EH, axis=1)
        w_f = pl.pallas_call(
            _filter_body,
            grid=(EH // BE,),
            in_specs=[
                pl.BlockSpec((G, BE), lambda i: (0, i)),
                pl.BlockSpec((1, 1, BE), lambda i: (i, 0, 0)),
                pl.BlockSpec((Fq, G), lambda i: (0, 0)),
                pl.BlockSpec((Fq, 1), lambda i: (0, 0)),
                pl.BlockSpec((Fq, Fq), lambda i: (0, 0)),
                pl.BlockSpec((Fq, 1), lambda i: (0, 0)),
            ],
            out_specs=pl.BlockSpec((BE, Fq // 2), lambda i: (i, 0)),
            out_shape=jax.ShapeDtypeStruct((EH, Fq // 2), jnp.int32),
        )(attr_h, ewh, mlp_w0, b0c, mlp_w1, b1c)
        src_h = lax.slice_in_dim(edge_index[0], lo, lo + EH)
        dst_h = lax.slice_in_dim(edge_index[1], lo, lo + EH)
        partials.append(
            _sc_gather_mul_scatter(xl, w_f, src_h, dst_h, zeros, K))

    out = pl.pallas_call(
        _final_body,
        out_shape=jax.ShapeDtypeStruct((N, H), jnp.float32),
    )(*partials, lin2_w, b2r)
    return out
